# baseline (device time: 17607 ns/iter reference)
import os

import jax
import jax.numpy as jnp
from jax import lax
from jax.experimental import pallas as pl
from jax.experimental.pallas import tpu as pltpu

N_DEV = 4
CHUNK_M = int(os.environ.get("KBLOCK", "256"))

_MODE = os.environ.get("KMODE", "full")


def kernel(x, dy, gamma):
    del gamma
    m, d = x.shape
    if m == 2048:
        sizes = [512, 512, 512, 256, 128, 64, 32, 32]
    else:
        sizes = [CHUNK_M] * (m // CHUNK_M)
    offsets = [sum(sizes[:i]) for i in range(len(sizes))]
    n_chunks = len(sizes)

    def body(x_hbm, dy_hbm, out_ref, xv, dyv, comm_ref,
             in_sems, send_sems, recv_sems):
        my_pos = lax.axis_index("i")

        copies = []
        for c in range(n_chunks):
            rows = pl.ds(offsets[c], sizes[c])
            cx = pltpu.make_async_copy(
                x_hbm.at[rows, :], xv.at[rows, :], in_sems.at[0, c])
            cy = pltpu.make_async_copy(
                dy_hbm.at[rows, :], dyv.at[rows, :], in_sems.at[1, c])
            cx.start()
            cy.start()
            copies.append((cx, cy))

        if _MODE == "full":
            barrier_sem = pltpu.get_barrier_semaphore()
            for k in range(1, N_DEV):
                pl.semaphore_signal(
                    barrier_sem, inc=1,
                    device_id=((my_pos + k) % N_DEV,),
                    device_id_type=pl.DeviceIdType.MESH,
                )
            pl.semaphore_wait(barrier_sem, N_DEV - 1)

        dgamma = jnp.zeros((1, d), jnp.float32)
        dbeta = jnp.zeros((1, d), jnp.float32)
        for c, (cx, cy) in enumerate(copies):
            cx.wait()
            cy.wait()
            rows = pl.ds(offsets[c], sizes[c])
            xb = xv[rows, :]
            dyb = dyv[rows, :]
            if _MODE == "dma_only":
                dgamma = dgamma + jnp.sum(xb, axis=0, keepdims=True)
                dbeta = dbeta + jnp.sum(dyb, axis=0, keepdims=True)
            else:
                mu = jnp.mean(xb, axis=1, keepdims=True)
                xc = xb - mu
                var = jnp.mean(xc * xc, axis=1, keepdims=True)
                rstd = lax.rsqrt(var + 1e-5)
                xhat = xc * rstd
                dgamma = dgamma + jnp.sum(dyb * xhat, axis=0, keepdims=True)
                dbeta = dbeta + jnp.sum(dyb, axis=0, keepdims=True)

        out_ref[0:1, :] = dgamma
        out_ref[1:2, :] = dbeta

        if _MODE == "full":
            comm_ref[my_pos, 0:1, :] = dgamma
            comm_ref[my_pos, 1:2, :] = dbeta
            sends = []
            for k in (2, 1, 3):
                peer = (my_pos + k) % N_DEV
                send = pltpu.make_async_remote_copy(
                    src_ref=comm_ref.at[my_pos],
                    dst_ref=comm_ref.at[my_pos],
                    send_sem=send_sems.at[k - 1],
                    recv_sem=recv_sems.at[my_pos],
                    device_id=(peer,),
                    device_id_type=pl.DeviceIdType.MESH,
                )
                send.start()
                sends.append(send)

            for k in (1, 3, 2):
                src = (my_pos + k) % N_DEV
                recv = pltpu.make_async_remote_copy(
                    src_ref=comm_ref.at[src],
                    dst_ref=comm_ref.at[src],
                    send_sem=send_sems.at[k - 1],
                    recv_sem=recv_sems.at[src],
                    device_id=(my_pos,),
                    device_id_type=pl.DeviceIdType.MESH,
                )
                recv.wait_recv()
                out_ref[...] += comm_ref[src]
            for send in sends:
                send.wait_send()

    return pl.pallas_call(
        body,
        in_specs=[
            pl.BlockSpec(memory_space=pl.ANY),
            pl.BlockSpec(memory_space=pl.ANY),
        ],
        out_specs=pl.BlockSpec(memory_space=pltpu.VMEM),
        out_shape=jax.ShapeDtypeStruct((2, d), jnp.float32),
        scratch_shapes=[
            pltpu.VMEM((m, d), jnp.float32),
            pltpu.VMEM((m, d), jnp.float32),
            pltpu.VMEM((N_DEV, 2, d), jnp.float32),
            pltpu.SemaphoreType.DMA((2, n_chunks)),
            pltpu.SemaphoreType.DMA((N_DEV - 1,)),
            pltpu.SemaphoreType.DMA((N_DEV,)),
        ],
        compiler_params=pltpu.CompilerParams(
            collective_id=0 if _MODE == "full" else None,
            vmem_limit_bytes=48 * 1024 * 1024,
        ),
    )(x, dy)


# device time: 17435 ns/iter; 1.0099x vs baseline; 1.0099x over previous
import os

import jax
import jax.numpy as jnp
from jax import lax
from jax.experimental import pallas as pl
from jax.experimental.pallas import tpu as pltpu

N_DEV = 4
CHUNK_M = int(os.environ.get("KBLOCK", "256"))

_MODE = os.environ.get("KMODE", "full")


def kernel(x, dy, gamma):
    del gamma
    m, d = x.shape
    sizes = [CHUNK_M] * (m // CHUNK_M)
    offsets = [sum(sizes[:i]) for i in range(len(sizes))]
    n_chunks = len(sizes)

    def body(x_hbm, dy_hbm, out_ref, xv, dyv, comm_ref,
             in_sems, send_sems, recv_sems):
        my_pos = lax.axis_index("i")

        copies = []
        for c in range(n_chunks):
            rows = pl.ds(offsets[c], sizes[c])
            cx = pltpu.make_async_copy(
                x_hbm.at[rows, :], xv.at[rows, :], in_sems.at[0, c])
            cy = pltpu.make_async_copy(
                dy_hbm.at[rows, :], dyv.at[rows, :], in_sems.at[1, c])
            cx.start()
            cy.start()
            copies.append((cx, cy))

        if _MODE == "full":
            barrier_sem = pltpu.get_barrier_semaphore()
            for k in range(1, N_DEV):
                pl.semaphore_signal(
                    barrier_sem, inc=1,
                    device_id=((my_pos + k) % N_DEV,),
                    device_id_type=pl.DeviceIdType.MESH,
                )
            pl.semaphore_wait(barrier_sem, N_DEV - 1)

        dgamma = jnp.zeros((1, d), jnp.float32)
        dbeta = jnp.zeros((1, d), jnp.float32)
        for c, (cx, cy) in enumerate(copies):
            cx.wait()
            cy.wait()
            rows = pl.ds(offsets[c], sizes[c])
            xb = xv[rows, :]
            dyb = dyv[rows, :]
            if _MODE == "dma_only":
                dgamma = dgamma + jnp.sum(xb, axis=0, keepdims=True)
                dbeta = dbeta + jnp.sum(dyb, axis=0, keepdims=True)
            else:
                mu = jnp.mean(xb, axis=1, keepdims=True)
                xc = xb - mu
                var = jnp.mean(xc * xc, axis=1, keepdims=True)
                rstd = lax.rsqrt(var + 1e-5)
                xhat = xc * rstd
                dgamma = dgamma + jnp.sum(dyb * xhat, axis=0, keepdims=True)
                dbeta = dbeta + jnp.sum(dyb, axis=0, keepdims=True)

        out_ref[0:1, :] = dgamma
        out_ref[1:2, :] = dbeta

        if _MODE == "full":
            comm_ref[my_pos, 0:1, :] = dgamma
            comm_ref[my_pos, 1:2, :] = dbeta
            sends = []
            for k in (2, 1, 3):
                peer = (my_pos + k) % N_DEV
                send = pltpu.make_async_remote_copy(
                    src_ref=comm_ref.at[my_pos],
                    dst_ref=comm_ref.at[my_pos],
                    send_sem=send_sems.at[k - 1],
                    recv_sem=recv_sems.at[my_pos],
                    device_id=(peer,),
                    device_id_type=pl.DeviceIdType.MESH,
                )
                send.start()
                sends.append(send)

            for k in (1, 3, 2):
                src = (my_pos + k) % N_DEV
                recv = pltpu.make_async_remote_copy(
                    src_ref=comm_ref.at[src],
                    dst_ref=comm_ref.at[src],
                    send_sem=send_sems.at[k - 1],
                    recv_sem=recv_sems.at[src],
                    device_id=(my_pos,),
                    device_id_type=pl.DeviceIdType.MESH,
                )
                recv.wait_recv()
                out_ref[...] += comm_ref[src]
            for send in sends:
                send.wait_send()

    return pl.pallas_call(
        body,
        in_specs=[
            pl.BlockSpec(memory_space=pl.ANY),
            pl.BlockSpec(memory_space=pl.ANY),
        ],
        out_specs=pl.BlockSpec(memory_space=pltpu.VMEM),
        out_shape=jax.ShapeDtypeStruct((2, d), jnp.float32),
        scratch_shapes=[
            pltpu.VMEM((m, d), jnp.float32),
            pltpu.VMEM((m, d), jnp.float32),
            pltpu.VMEM((N_DEV, 2, d), jnp.float32),
            pltpu.SemaphoreType.DMA((2, n_chunks)),
            pltpu.SemaphoreType.DMA((N_DEV - 1,)),
            pltpu.SemaphoreType.DMA((N_DEV,)),
        ],
        compiler_params=pltpu.CompilerParams(
            collective_id=0 if _MODE == "full" else None,
            vmem_limit_bytes=48 * 1024 * 1024,
        ),
    )(x, dy)
